# baseline (device time: 319480 ns/iter reference)
import jax
import jax.numpy as jnp
from jax import lax
from jax.experimental import pallas as pl
from jax.experimental.pallas import tpu as pltpu

N_DEV = 4
M_PER = 1024
N_TOTAL = 8192
N_BLK = 512
N_ROUNDS = 4
CCW0 = N_TOTAL // 2

RECV_OFF_CW = (2, 1, 0)
RECV_OFF_CCW = (2, 3, 0)


def kernel(x, w_mat):
    x16 = x.astype(jnp.bfloat16)

    def body(x_ref, w_ref, out_ref,
             comm_cw, comm_ccw, acc, w16, wstage,
             send_cw, recv_cw, send_ccw, recv_ccw, out_sems, wsems,
             credit_cw, credit_ccw):
        my_pos = lax.axis_index("i")
        left = lax.rem(my_pos + (N_DEV - 1), N_DEV)
        right = lax.rem(my_pos + 1, N_DEV)

        barrier_sem = pltpu.get_barrier_semaphore()
        for nbr in (left, right):
            pl.semaphore_signal(
                barrier_sem, inc=1,
                device_id=(nbr,), device_id_type=pl.DeviceIdType.MESH,
            )
        pl.semaphore_wait(barrier_sem, 2)

        def cols(cw, r, lane):
            return (0 if cw else CCW0) + (2 * r + lane) * N_BLK

        def wslot(r, cw, lane):
            return (r % 2) * 4 + (0 if cw else 2) + lane

        def fetch_w(r):
            blocks = [(cw, lane) for cw in (True, False) for lane in (0, 1)]
            dmas = {}
            for i, (cw, lane) in enumerate(blocks):
                st = i % 2
                if i >= 2:
                    dmas[i - 2].wait()
                    pcw, plane = blocks[i - 2]
                    w16[wslot(r, pcw, plane)] = (
                        wstage[st][...].astype(jnp.bfloat16))
                d = pltpu.make_async_copy(
                    w_ref.at[:, pl.ds(cols(cw, r, lane), N_BLK)],
                    wstage.at[st], wsems.at[st])
                d.start()
                dmas[i] = d
            for i in (2, 3):
                dmas[i].wait()
                pcw, plane = blocks[i]
                w16[wslot(r, pcw, plane)] = (
                    wstage[i % 2][...].astype(jnp.bfloat16))

        def partial(off, ws):
            row0 = lax.rem(my_pos + off, N_DEV) * M_PER
            return jnp.dot(
                x_ref[pl.ds(row0, M_PER), :],
                w16[ws],
                preferred_element_type=jnp.float32,
            )

        def hop(cw, lane, h):
            comm = comm_cw if cw else comm_ccw
            base = 2 * lane
            return pltpu.make_async_remote_copy(
                src_ref=comm.at[base + h % 2],
                dst_ref=comm.at[base + (h + 1) % 2],
                send_sem=(send_cw if cw else send_ccw).at[base + h % 2],
                recv_sem=(recv_cw if cw else recv_ccw).at[base + (h + 1) % 2],
                device_id=(right if cw else left,),
                device_id_type=pl.DeviceIdType.MESH,
            )

        def add_into(cw, slot, p):
            comm = comm_cw if cw else comm_ccw
            comm[slot] = comm[slot][...] + p.astype(jnp.bfloat16)

        def load_payload(r, cw, lane):
            comm = comm_cw if cw else comm_ccw
            off = 3 if cw else 1
            comm[2 * lane] = partial(off, wslot(r, cw, lane)).astype(
                jnp.bfloat16)

        def wait_out(r):
            for cw in (True, False):
                for lane in (0, 1):
                    a = 2 * lane + (0 if cw else 1)
                    pltpu.make_async_copy(
                        acc.at[a],
                        out_ref.at[:, pl.ds(cols(cw, r, lane), N_BLK)],
                        out_sems.at[a],
                    ).wait()

        rd = {}
        fetch_w(0)
        for lane in (0, 1):
            load_payload(0, True, lane)
            load_payload(0, False, lane)
            for cw in (True, False):
                rd[(cw, lane)] = hop(cw, lane, 0)
                rd[(cw, lane)].start()

        for r in range(N_ROUNDS):
            c = {(cw, lane): cols(cw, r, lane)
                 for cw in (True, False) for lane in (0, 1)}

            if r > 0:
                wait_out(r - 1)
            if r + 1 < N_ROUNDS:
                fetch_w(r + 1)

            for h in range(N_DEV - 1):
                last = h == N_DEV - 2
                for lane in (0, 1):
                    p_cw = partial(RECV_OFF_CW[h], wslot(r, True, lane))
                    p_ccw = partial(RECV_OFF_CCW[h], wslot(r, False, lane))
                    for cw, pp in ((True, p_cw), (False, p_ccw)):
                        rdma = rd[(cw, lane)]
                        rdma.wait()
                        rslot = 2 * lane + (h + 1) % 2
                        if not last:
                            add_into(cw, rslot, pp)
                            rd[(cw, lane)] = hop(cw, lane, h + 1)
                            rd[(cw, lane)].start()
                        else:
                            comm = comm_cw if cw else comm_ccw
                            a = 2 * lane + (0 if cw else 1)
                            acc[a] = comm[rslot][...].astype(
                                jnp.float32) + pp
                            pltpu.make_async_copy(
                                acc.at[a],
                                out_ref.at[:, pl.ds(c[(cw, lane)], N_BLK)],
                                out_sems.at[a],
                            ).start()
                    if last and r + 1 < N_ROUNDS:
                        pl.semaphore_signal(
                            credit_cw.at[lane], inc=1, device_id=(left,),
                            device_id_type=pl.DeviceIdType.MESH)
                        pl.semaphore_signal(
                            credit_ccw.at[lane], inc=1, device_id=(right,),
                            device_id_type=pl.DeviceIdType.MESH)
                        load_payload(r + 1, True, lane)
                        load_payload(r + 1, False, lane)
                        pl.semaphore_wait(credit_cw.at[lane], 1)
                        pl.semaphore_wait(credit_ccw.at[lane], 1)
                        for cw in (True, False):
                            rd[(cw, lane)] = hop(cw, lane, 0)
                            rd[(cw, lane)].start()

        wait_out(N_ROUNDS - 1)

    out = pl.pallas_call(
        body,
        out_shape=jax.ShapeDtypeStruct((M_PER, N_TOTAL), jnp.float32),
        in_specs=[
            pl.BlockSpec(memory_space=pltpu.VMEM),
            pl.BlockSpec(memory_space=pl.ANY),
        ],
        out_specs=pl.BlockSpec(memory_space=pl.ANY),
        scratch_shapes=[
            pltpu.VMEM((4, M_PER, N_BLK), jnp.bfloat16),
            pltpu.VMEM((4, M_PER, N_BLK), jnp.bfloat16),
            pltpu.VMEM((4, M_PER, N_BLK), jnp.float32),
            pltpu.VMEM((8, M_PER, N_BLK), jnp.bfloat16),
            pltpu.VMEM((2, M_PER, N_BLK), jnp.float32),
            pltpu.SemaphoreType.DMA((4,)),
            pltpu.SemaphoreType.DMA((4,)),
            pltpu.SemaphoreType.DMA((4,)),
            pltpu.SemaphoreType.DMA((4,)),
            pltpu.SemaphoreType.DMA((4,)),
            pltpu.SemaphoreType.DMA((2,)),
            pltpu.SemaphoreType.REGULAR((2,)),
            pltpu.SemaphoreType.REGULAR((2,)),
        ],
        compiler_params=pltpu.CompilerParams(
            collective_id=0,
            vmem_limit_bytes=50 * 1024 * 1024,
        ),
    )(x16, w_mat)
    return out
